# Initial kernel scaffold; baseline (speedup 1.0000x reference)
import jax
import jax.numpy as jnp
from jax.experimental import pallas as pl


def kernel(ori_x, ori_xyz, sub_x, sub_xyz, W1, g1, b1, W2, g2, b2):
    raise NotImplementedError("write your pallas kernel here")


# speedup vs baseline: 12.3441x; 12.3441x over previous
import functools

import jax
import jax.numpy as jnp
from jax import lax
from jax.experimental import pallas as pl
from jax.experimental.pallas import tpu as pltpu
from jax.experimental.pallas import tpu_sc as plsc

_B, _C, _N, _M, _K = 16, 256, 4096, 1024, 3
_O = 512
_EPS = 1e-5
_BN = _B * _N

_NBQ = 512
_NBX = 512

_NW = 32
_CHUNK = 64
_ROWS_W = _BN // _NW
_NCH = _ROWS_W // _CHUNK


def _knn_body(q_ref, s_ref, idx_ref, w_ref):
    b = pl.program_id(0)
    q = q_ref[0]
    s = s_ref[0]
    qn = jnp.sum(q * q, axis=0)
    sn = jnp.sum(s * s, axis=0)
    cross = lax.dot_general(q, s, (((0,), (0,)), ((), ())),
                            preferred_element_type=jnp.float32)
    d2 = qn[:, None] + sn[None, :] - 2.0 * cross
    cols = lax.broadcasted_iota(jnp.int32, (_NBQ, _M), 1)
    idxs, dks = [], []
    for _ in range(_K):
        dmin = jnp.min(d2, axis=1)
        hit = d2 <= dmin[:, None]
        amin = jnp.min(jnp.where(hit, cols, _M), axis=1)
        d2 = jnp.where(cols == amin[:, None], jnp.inf, d2)
        idxs.append(amin)
        dks.append(jnp.sqrt(jnp.maximum(dmin, 1e-12)))
    w = jnp.stack([1.0 / (dk + _EPS) for dk in dks], axis=0)
    w = w / jnp.sum(w, axis=0, keepdims=True)
    idx_ref[...] = jnp.stack(idxs, axis=0) + b * _M
    w_ref[...] = w


def _knn(qpad, spad):
    nblk = _N // _NBQ
    return pl.pallas_call(
        _knn_body,
        grid=(_B, nblk),
        in_specs=[
            pl.BlockSpec((1, 8, _NBQ), lambda b, n: (b, 0, n)),
            pl.BlockSpec((1, 8, _M), lambda b, n: (b, 0, 0)),
        ],
        out_specs=[
            pl.BlockSpec((_K, _NBQ), lambda b, n: (0, b * nblk + n)),
            pl.BlockSpec((_K, _NBQ), lambda b, n: (0, b * nblk + n)),
        ],
        out_shape=[
            jax.ShapeDtypeStruct((_K, _BN), jnp.int32),
            jax.ShapeDtypeStruct((_K, _BN), jnp.float32),
        ],
    )(qpad, spad)


def _gather(table, fidx, wts):
    mesh = plsc.VectorSubcoreMesh(core_axis_name="c", subcore_axis_name="s")

    @functools.partial(
        pl.kernel, mesh=mesh,
        out_type=jax.ShapeDtypeStruct((_BN, _C), jnp.float32),
        scratch_types=[
            pltpu.VMEM((_CHUNK,), jnp.int32),
            pltpu.VMEM((_CHUNK,), jnp.int32),
            pltpu.VMEM((_CHUNK,), jnp.int32),
            pltpu.VMEM((_CHUNK,), jnp.float32),
            pltpu.VMEM((_CHUNK,), jnp.float32),
            pltpu.VMEM((_CHUNK,), jnp.float32),
            pltpu.VMEM((_CHUNK, _C), jnp.float32),
            pltpu.VMEM((_CHUNK, _C), jnp.float32),
            pltpu.VMEM((_CHUNK, _C), jnp.float32),
            pltpu.VMEM((_CHUNK, _C), jnp.float32),
            pltpu.SemaphoreType.DMA,
        ],
    )
    def k(table_hbm, idx_hbm, w_hbm, out_hbm,
          i0, i1, i2, w0v, w1v, w2v, g0, g1, g2, ov, sem):
        wid = lax.axis_index("s") * 2 + lax.axis_index("c")
        base = wid * _ROWS_W

        def chunk(c, carry):
            off = base + c * _CHUNK
            sl = pl.ds(off, _CHUNK)
            pltpu.sync_copy(idx_hbm.at[0, sl], i0)
            pltpu.sync_copy(idx_hbm.at[1, sl], i1)
            pltpu.sync_copy(idx_hbm.at[2, sl], i2)
            pltpu.sync_copy(w_hbm.at[0, sl], w0v)
            pltpu.sync_copy(w_hbm.at[1, sl], w1v)
            pltpu.sync_copy(w_hbm.at[2, sl], w2v)
            cp0 = pltpu.async_copy(table_hbm.at[i0], g0, sem)
            cp1 = pltpu.async_copy(table_hbm.at[i1], g1, sem)
            cp2 = pltpu.async_copy(table_hbm.at[i2], g2, sem)
            cp0.wait()
            cp1.wait()
            cp2.wait()

            def grp(g, cr):
                wv0 = w0v[pl.ds(g * 16, 16)]
                wv1 = w1v[pl.ds(g * 16, 16)]
                wv2 = w2v[pl.ds(g * 16, 16)]
                for r16 in range(16):
                    r = g * 16 + r16
                    w0 = wv0[r16]
                    w1 = wv1[r16]
                    w2 = wv2[r16]
                    for j in range(_C // 16):
                        cs = pl.ds(j * 16, 16)
                        ov[r, cs] = (g0[r, cs] * w0 + g1[r, cs] * w1
                                     + g2[r, cs] * w2)
                return cr

            lax.fori_loop(0, _CHUNK // 16, grp, 0)
            pltpu.sync_copy(ov, out_hbm.at[sl])
            return carry

        lax.fori_loop(0, _NCH, chunk, 0)

    return k(table, fidx, wts)


def _gelu(x):
    return 0.5 * x * (1.0 + lax.erf(x * 0.7071067811865476))


def _partials(y):
    s = jnp.sum(y, axis=1)
    ss = jnp.sum(y * y, axis=1)
    return jnp.concatenate([s[None], ss[None],
                            jnp.zeros((6, _O), jnp.float32)], axis=0)


def _mlp1_body(ox_ref, ng_ref, w_ref, y_ref, ps_ref):
    ox = ox_ref[0]
    ng = ng_ref[0]
    w = w_ref[...]
    y = (lax.dot_general(w[:, :_C], ox, (((1,), (0,)), ((), ())),
                         preferred_element_type=jnp.float32)
         + lax.dot_general(w[:, _C:], ng, (((1,), (1,)), ((), ())),
                           preferred_element_type=jnp.float32))
    y_ref[0] = y
    ps_ref[0] = _partials(y)


def _mlp1(ori_x, neigh, W1):
    nblk = _N // _NBX
    return pl.pallas_call(
        _mlp1_body,
        grid=(_B, nblk),
        in_specs=[
            pl.BlockSpec((1, _C, _NBX), lambda b, n: (b, 0, n)),
            pl.BlockSpec((1, _NBX, _C), lambda b, n: (b, n, 0)),
            pl.BlockSpec((_O, 2 * _C), lambda b, n: (0, 0)),
        ],
        out_specs=[
            pl.BlockSpec((1, _O, _NBX), lambda b, n: (b, 0, n)),
            pl.BlockSpec((1, 8, _O), lambda b, n: (b * nblk + n, 0, 0)),
        ],
        out_shape=[
            jax.ShapeDtypeStruct((_B, _O, _N), jnp.float32),
            jax.ShapeDtypeStruct((_B * nblk, 8, _O), jnp.float32),
        ],
    )(ori_x, neigh, W1)


def _mlp2_body(y1_ref, p_ref, w_ref, y_ref, ps_ref):
    p = p_ref[...]
    h = (y1_ref[0] - p[:, 0:1]) * p[:, 1:2] * p[:, 2:3] + p[:, 3:4]
    h = _gelu(h)
    y = lax.dot_general(w_ref[...], h, (((1,), (0,)), ((), ())),
                        preferred_element_type=jnp.float32)
    y_ref[0] = y
    ps_ref[0] = _partials(y)


def _mlp2(y1, p1, W2):
    nblk = _N // _NBX
    return pl.pallas_call(
        _mlp2_body,
        grid=(_B, nblk),
        in_specs=[
            pl.BlockSpec((1, _O, _NBX), lambda b, n: (b, 0, n)),
            pl.BlockSpec((_O, 8), lambda b, n: (0, 0)),
            pl.BlockSpec((_O, _O), lambda b, n: (0, 0)),
        ],
        out_specs=[
            pl.BlockSpec((1, _O, _NBX), lambda b, n: (b, 0, n)),
            pl.BlockSpec((1, 8, _O), lambda b, n: (b * nblk + n, 0, 0)),
        ],
        out_shape=[
            jax.ShapeDtypeStruct((_B, _O, _N), jnp.float32),
            jax.ShapeDtypeStruct((_B * nblk, 8, _O), jnp.float32),
        ],
    )(y1, p1, W2)


def _norm_body(y_ref, p_ref, o_ref):
    p = p_ref[...]
    h = (y_ref[0] - p[:, 0:1]) * p[:, 1:2] * p[:, 2:3] + p[:, 3:4]
    o_ref[0] = _gelu(h)


def _norm(y2, p2):
    return pl.pallas_call(
        _norm_body,
        grid=(_B, _N // _NBX),
        in_specs=[
            pl.BlockSpec((1, _O, _NBX), lambda b, n: (b, 0, n)),
            pl.BlockSpec((_O, 8), lambda b, n: (0, 0)),
        ],
        out_specs=pl.BlockSpec((1, _O, _NBX), lambda b, n: (b, 0, n)),
        out_shape=jax.ShapeDtypeStruct((_B, _O, _N), jnp.float32),
    )(y2, p2)


def _bn_params(ps, g, b):
    cnt = float(_BN)
    mu = jnp.sum(ps[:, 0, :], axis=0) / cnt
    var = jnp.maximum(jnp.sum(ps[:, 1, :], axis=0) / cnt - mu * mu, 0.0)
    rstd = 1.0 / jnp.sqrt(var + 1e-5)
    z = jnp.zeros_like(g)
    return jnp.stack([mu, rstd, g, b, z, z, z, z], axis=1)


def kernel(ori_x, ori_xyz, sub_x, sub_xyz, W1, g1, b1, W2, g2, b2):
    qpad = jnp.pad(ori_xyz, ((0, 0), (0, 5), (0, 0)))
    spad = jnp.pad(sub_xyz, ((0, 0), (0, 5), (0, 0)))
    fidx, wts = _knn(qpad, spad)
    table = jnp.transpose(sub_x, (0, 2, 1)).reshape(_B * _M, _C)
    neigh = _gather(table, fidx, wts).reshape(_B, _N, _C)
    y1, ps1 = _mlp1(ori_x, neigh, W1)
    y2, ps2 = _mlp2(y1, _bn_params(ps1, g1, b1), W2)
    return _norm(y2, _bn_params(ps2, g2, b2))
